# baseline (speedup 1.0000x reference)
import functools

import jax
import jax.numpy as jnp
from jax import lax
from jax.experimental import pallas as pl
from jax.experimental.pallas import tpu as pltpu
from jax.experimental.pallas import tpu_sc as plsc

_NC = 2
_NS = 16
_NW = _NC * _NS
_LANES = 16
_R = 16
_NSLOT = 5
_AHEAD = 3


def _make_sc_add(B, T, C):
    t_per_w = T // _NW
    nchunks = t_per_w // _R
    nunits = nchunks * B
    shift = C.bit_length() - 1
    assert C == 1 << shift

    mesh = plsc.VectorSubcoreMesh(core_axis_name="c", subcore_axis_name="s")

    @functools.partial(
        pl.kernel,
        mesh=mesh,
        out_type=jax.ShapeDtypeStruct((B * T, C), jnp.float32),
        scratch_types=[
            [pltpu.VMEM((_R, C), jnp.float32) for _ in range(2)],
            [pltpu.VMEM((_R, C), jnp.float32) for _ in range(_NSLOT)],
            [pltpu.SemaphoreType.DMA for _ in range(2)],
            [pltpu.SemaphoreType.DMA for _ in range(_NSLOT)],
            [pltpu.SemaphoreType.DMA for _ in range(_NSLOT)],
        ],
    )
    def sc_add(x_hbm, pos_hbm, out_hbm, pos_v, x_v, psem, xsem, ssem):
        wid = lax.axis_index("s") * _NC + lax.axis_index("c")
        base_t = wid * t_per_w

        def pos_row(i):
            return base_t + i * _R

        def x_row(u):
            i, b = divmod(u, B)
            return b * T + pos_row(i)

        def load_x(u):
            s = u % _NSLOT
            return pltpu.async_copy(
                x_hbm.at[pl.ds(x_row(u), _R)], x_v[s], xsem[s]
            )

        hpos = {0: pltpu.async_copy(pos_hbm.at[pl.ds(pos_row(0), _R)],
                                    pos_v[0], psem[0])}
        hx = {u: load_x(u) for u in range(_AHEAD)}
        hs = {}
        for u in range(nunits):
            i = u // B
            if u % B == 0:
                hpos.pop(i % 2).wait()
                if i + 1 < nchunks:
                    hpos[(i + 1) % 2] = pltpu.async_copy(
                        pos_hbm.at[pl.ds(pos_row(i + 1), _R)],
                        pos_v[(i + 1) % 2], psem[(i + 1) % 2],
                    )
            nxt = u + _AHEAD
            if nxt < nunits:
                if nxt - _NSLOT >= 0:
                    hs.pop(nxt % _NSLOT).wait()
                hx[nxt] = load_x(nxt)

            hx.pop(u).wait()
            s = u % _NSLOT
            xs, ps = x_v[s], pos_v[i % 2]

            @plsc.parallel_loop(0, _R * C, step=_LANES, unroll=8)
            def _add(j):
                r = lax.shift_right_logical(j, shift)
                c = pl.multiple_of(lax.bitwise_and(j, C - 1), _LANES)
                xs[r, pl.ds(c, _LANES)] = (
                    xs[r, pl.ds(c, _LANES)] + ps[r, pl.ds(c, _LANES)]
                )

            hs[s] = pltpu.async_copy(
                x_v[s], out_hbm.at[pl.ds(x_row(u), _R)], ssem[s]
            )
        for s in sorted(hs):
            hs.pop(s).wait()

    return sc_add


def kernel(x, pos_embedding):
    B, T, C = x.shape
    sc_add = _make_sc_add(B, T, C)
    out = sc_add(x.reshape(B * T, C), pos_embedding)
    return out.reshape(B, T, C)
